# baseline (speedup 1.0000x reference)
import functools

import jax
import jax.numpy as jnp
from jax import lax
from jax.experimental import pallas as pl
from jax.experimental.pallas import tpu as pltpu
from jax.experimental.pallas import tpu_sc as plsc

_MAX_VAL = 52000.0
_MIN_VAL = -53000.0
_NUM_LEVELS = 1000
_C = 24
_T = 256
_D = 4096
_LANES = 16
_NCH = 3
_TSUB = 8
_NSUB = 8


def _quant(x):
    y = (x - _MIN_VAL) / (_MAX_VAL - _MIN_VAL) * (_NUM_LEVELS - 1)
    return jnp.clip(jnp.round(y), 0, _NUM_LEVELS - 1).astype(jnp.int32)


def _idx_body(in_ref, out_ref):
    out_ref[...] = _quant(in_ref[...])


def _roll_lanes(x, shift):
    return jnp.concatenate([x[:, -shift:], x[:, :-shift]], axis=1)


def _fin_body(p_ref, out_ref):
    s = jnp.sum(p_ref[...], axis=1)
    qa = jnp.where(s > 0, 1.0, -1.0)
    r3 = _roll_lanes(qa, 3)
    r2 = _roll_lanes(qa, 2)
    r1 = _roll_lanes(qa, 1)
    w = (r3[0 : _C - 3] * r2[1 : _C - 2]) * (r1[2 : _C - 1] * qa[3:_C])
    s2 = jnp.sum(w, axis=0, keepdims=True)
    out_ref[...] = jnp.where(s2 > 0, 1.0, -1.0)


@functools.partial(
    pl.kernel,
    mesh=plsc.VectorSubcoreMesh(core_axis_name="c", subcore_axis_name="s"),
    out_type=jax.ShapeDtypeStruct((_C * 4, _D), jnp.float32),
    scratch_types=[
        pltpu.VMEM((_TSUB,), jnp.int32),
        pltpu.VMEM((_TSUB, _D), jnp.float32),
        pltpu.VMEM((_TSUB, _D), jnp.float32),
        pltpu.VMEM((_NCH, _D), jnp.float32),
        pltpu.SemaphoreType.DMA,
    ],
)
def _sc_stage(idx_hbm, L_hbm, F_hbm, out_hbm, idxv, Fv, rows, acc, sem):
    wid = lax.axis_index("s") * 2 + lax.axis_index("c")
    cg = lax.rem(wid, 8)
    tq = wid // 8
    c0 = cg * _NCH
    t0 = tq * 64

    zero = jnp.zeros((_LANES,), jnp.float32)

    def _zero_body(i, _):
        for j in range(_NCH):
            acc[j, pl.ds(i * _LANES, _LANES)] = zero
        return 0

    lax.fori_loop(0, _D // _LANES, _zero_body, 0)

    def _chunk(k, _):
        pltpu.sync_copy(F_hbm.at[pl.ds(t0 + k * _TSUB, _TSUB)], Fv)
        for j in range(_NCH):
            base = (c0 + j) * _T + t0 + k * _TSUB
            pltpu.sync_copy(idx_hbm.at[pl.ds(base, _TSUB)], idxv)
            pltpu.async_copy(L_hbm.at[idxv], rows, sem).wait()

            def _accum(dc, _, j=j):
                dsl = pl.ds(dc * _LANES, _LANES)
                a = acc[j, dsl]
                for t in range(_TSUB):
                    a = a + rows[t, dsl] * Fv[t, dsl]
                acc[j, dsl] = a
                return 0

            lax.fori_loop(0, _D // _LANES, _accum, 0)
        return 0

    lax.fori_loop(0, _NSUB, _chunk, 0)

    for j in range(_NCH):
        pltpu.sync_copy(
            acc.at[pl.ds(j, 1)], out_hbm.at[pl.ds((c0 + j) * 4 + tq, 1)]
        )


@jax.jit
def kernel(input, level_weight, features_weight):
    idx = pl.pallas_call(
        _idx_body,
        out_shape=jax.ShapeDtypeStruct((_C, _T), jnp.int32),
    )(input)
    idx_flat = jnp.reshape(idx, (_C * _T,))
    partials = _sc_stage(idx_flat, level_weight, features_weight)
    p4 = jnp.reshape(partials, (_C, 4, _D))
    out = pl.pallas_call(
        _fin_body,
        out_shape=jax.ShapeDtypeStruct((1, _D), jnp.float32),
    )(p4)
    return out


# speedup vs baseline: 1.9711x; 1.9711x over previous
import functools

import jax
import jax.numpy as jnp
from jax import lax
from jax.experimental import pallas as pl
from jax.experimental.pallas import tpu as pltpu
from jax.experimental.pallas import tpu_sc as plsc

_MAX_VAL = 52000.0
_MIN_VAL = -53000.0
_NUM_LEVELS = 1000
_C = 24
_C_SC = 8
_C_TC = _C - _C_SC
_T = 256
_P = _T // 2
_D = 4096
_W = 128.0
_LANES = 16
_TSUB = 8
_NSUB = 8


def _quant(x):
    y = (x - _MIN_VAL) / (_MAX_VAL - _MIN_VAL) * (_NUM_LEVELS - 1)
    return jnp.clip(jnp.round(y), 0, _NUM_LEVELS - 1).astype(jnp.int32)


def _roll_lanes(x, shift):
    return jnp.concatenate([x[:, -shift:], x[:, :-shift]], axis=1)


def _idx_body(in_ref, out_ref):
    out_ref[...] = _quant(in_ref[...])


@functools.partial(
    pl.kernel,
    mesh=plsc.VectorSubcoreMesh(core_axis_name="c", subcore_axis_name="s"),
    out_type=jax.ShapeDtypeStruct((_C_SC * 4, _D), jnp.float32),
    scratch_types=[
        pltpu.VMEM((_TSUB,), jnp.int32),
        pltpu.VMEM((_TSUB, _D), jnp.float32),
        pltpu.VMEM((_TSUB, _D), jnp.float32),
        pltpu.VMEM((1, _D), jnp.float32),
        pltpu.SemaphoreType.DMA,
    ],
)
def _sc_stage(idx_hbm, L_hbm, F_hbm, out_hbm, idxv, Fv, rows, acc, sem):
    wid = lax.axis_index("s") * 2 + lax.axis_index("c")
    c0 = lax.rem(wid, _C_SC)
    tq = wid // _C_SC
    t0 = tq * 64

    zero = jnp.zeros((_LANES,), jnp.float32)

    def _zero_body(i, _):
        acc[0, pl.ds(i * _LANES, _LANES)] = zero
        return 0

    lax.fori_loop(0, _D // _LANES, _zero_body, 0)

    def _chunk(k, _):
        pltpu.sync_copy(F_hbm.at[pl.ds(t0 + k * _TSUB, _TSUB)], Fv)
        base = c0 * _T + t0 + k * _TSUB
        pltpu.sync_copy(idx_hbm.at[pl.ds(base, _TSUB)], idxv)
        pltpu.async_copy(L_hbm.at[idxv], rows, sem).wait()

        def _accum(dc, _):
            dsl = pl.ds(dc * _LANES, _LANES)
            a = acc[0, dsl]
            for t in range(_TSUB):
                a = a + rows[t, dsl] * Fv[t, dsl]
            acc[0, dsl] = a
            return 0

        lax.fori_loop(0, _D // _LANES, _accum, 0)
        return 0

    lax.fori_loop(0, _NSUB, _chunk, 0)

    pltpu.sync_copy(acc.at[pl.ds(0, 1)], out_hbm.at[pl.ds(c0 * 4 + tq, 1)])


def _tc_body(in_ref, L_ref, F_ref, out_ref, Lbf_ref, Gm_ref, Fo_ref):
    c = pl.program_id(0)

    @pl.when(c == 0)
    def _():
        Lbf_ref[...] = L_ref[...].astype(jnp.bfloat16)
        fo = F_ref[:, 1, :] * _W
        Fo_ref[...] = fo
        Gm_ref[...] = F_ref[:, 0, :] - fo

    idx_e = _quant(in_ref[0, :, 0:1])
    idx_o = _quant(in_ref[0, :, 1:2])
    lvl = jax.lax.broadcasted_iota(jnp.int32, (_P, _NUM_LEVELS), 1)
    oh = (idx_e == lvl).astype(jnp.bfloat16) + (idx_o == lvl).astype(
        jnp.bfloat16
    ) * jnp.bfloat16(1.0 / _W)
    g = jnp.dot(oh, Lbf_ref[...], preferred_element_type=jnp.float32)
    mask = g > 0
    s = jnp.sum(jnp.where(mask, Gm_ref[...], -Gm_ref[...]) + g * Fo_ref[...],
                axis=0, keepdims=True)
    out_ref[0] = jnp.where(s > 0, 1.0, -1.0)


def _fin_body(p_ref, q_ref, out_ref):
    s = jnp.sum(p_ref[...], axis=1)
    q_sc = jnp.where(s > 0, 1.0, -1.0)
    qa = jnp.concatenate([q_sc, q_ref[...]], axis=0)
    r3 = _roll_lanes(qa, 3)
    r2 = _roll_lanes(qa, 2)
    r1 = _roll_lanes(qa, 1)
    w = (r3[0 : _C - 3] * r2[1 : _C - 2]) * (r1[2 : _C - 1] * qa[3:_C])
    s2 = jnp.sum(w, axis=0, keepdims=True)
    out_ref[...] = jnp.where(s2 > 0, 1.0, -1.0)


@jax.jit
def kernel(input, level_weight, features_weight):
    idx = pl.pallas_call(
        _idx_body,
        out_shape=jax.ShapeDtypeStruct((_C_SC, _T), jnp.int32),
    )(input[:_C_SC])
    idx_flat = jnp.reshape(idx, (_C_SC * _T,))
    partials = _sc_stage(idx_flat, level_weight, features_weight)

    x3 = jnp.reshape(input[_C_SC:], (_C_TC, _P, 2))
    F3 = jnp.reshape(features_weight, (_P, 2, _D))
    q_tc = pl.pallas_call(
        _tc_body,
        grid=(_C_TC,),
        in_specs=[
            pl.BlockSpec((1, _P, 2), lambda c: (c, 0, 0)),
            pl.BlockSpec((_NUM_LEVELS, _D), lambda c: (0, 0)),
            pl.BlockSpec((_P, 2, _D), lambda c: (0, 0, 0)),
        ],
        out_specs=pl.BlockSpec((1, 1, _D), lambda c: (c, 0, 0)),
        out_shape=jax.ShapeDtypeStruct((_C_TC, 1, _D), jnp.float32),
        scratch_shapes=[
            pltpu.VMEM((_NUM_LEVELS, _D), jnp.bfloat16),
            pltpu.VMEM((_P, _D), jnp.float32),
            pltpu.VMEM((_P, _D), jnp.float32),
        ],
    )(x3, level_weight, F3)

    p4 = jnp.reshape(partials, (_C_SC, 4, _D))
    out = pl.pallas_call(
        _fin_body,
        out_shape=jax.ShapeDtypeStruct((1, _D), jnp.float32),
    )(p4, jnp.reshape(q_tc, (_C_TC, _D)))
    return out


# speedup vs baseline: 2.0410x; 1.0355x over previous
import functools

import jax
import jax.numpy as jnp
from jax import lax
from jax.experimental import pallas as pl
from jax.experimental.pallas import tpu as pltpu
from jax.experimental.pallas import tpu_sc as plsc

_MAX_VAL = 52000.0
_MIN_VAL = -53000.0
_NUM_LEVELS = 1000
_C = 24
_C_SC = 8
_C_TC = _C - _C_SC
_T = 256
_P = _T // 2
_D = 4096
_W = 128.0
_LANES = 16
_TSUB = 8
_NSUB = 8


def _quant(x):
    y = (x - _MIN_VAL) / (_MAX_VAL - _MIN_VAL) * (_NUM_LEVELS - 1)
    return jnp.clip(jnp.round(y), 0, _NUM_LEVELS - 1).astype(jnp.int32)


def _roll_lanes(x, shift):
    return jnp.concatenate([x[:, -shift:], x[:, :-shift]], axis=1)


def _idx_body(in_ref, out_ref):
    out_ref[...] = _quant(in_ref[...])


@functools.partial(
    pl.kernel,
    mesh=plsc.VectorSubcoreMesh(core_axis_name="c", subcore_axis_name="s"),
    out_type=jax.ShapeDtypeStruct((_C_SC * 4, _D), jnp.float32),
    scratch_types=[
        pltpu.VMEM((64,), jnp.int32),
        pltpu.VMEM((_TSUB, _D), jnp.float32),
        pltpu.VMEM((_TSUB, _D), jnp.float32),
        pltpu.VMEM((_TSUB, _D), jnp.float32),
        pltpu.VMEM((1, _D), jnp.float32),
        pltpu.SemaphoreType.DMA,
        pltpu.SemaphoreType.DMA,
        pltpu.SemaphoreType.DMA,
    ],
)
def _sc_stage(idx_hbm, L_hbm, F_hbm, out_hbm, idxv, fv, r0, r1, acc,
              sf, sg0, sg1):
    wid = lax.axis_index("s") * 2 + lax.axis_index("c")
    c0 = lax.rem(wid, _C_SC)
    tq = wid // _C_SC
    t0 = tq * 64

    pltpu.sync_copy(idx_hbm.at[pl.ds(c0 * _T + t0, 64)], idxv)

    zero = jnp.zeros((_LANES,), jnp.float32)

    def _zero_body(i, _):
        acc[0, pl.ds(i * _LANES, _LANES)] = zero
        return 0

    lax.fori_loop(0, _D // _LANES, _zero_body, 0)

    rbuf = (r0, r1)
    gsem = (sg0, sg1)

    def _start_g(k, b):
        ks = jnp.minimum(k, _NSUB - 1)
        pltpu.async_copy(
            L_hbm.at[idxv.at[pl.ds(ks * _TSUB, _TSUB)]], rbuf[b], gsem[b]
        )

    def _start_f(k):
        ks = jnp.minimum(k, _NSUB - 1)
        pltpu.async_copy(F_hbm.at[pl.ds(t0 + ks * _TSUB, _TSUB)], fv, sf)

    def _wait_g(b):
        pltpu.make_async_copy(
            L_hbm.at[idxv.at[pl.ds(0, _TSUB)]], rbuf[b], gsem[b]
        ).wait()

    def _wait_f():
        pltpu.make_async_copy(F_hbm.at[pl.ds(t0, _TSUB)], fv, sf).wait()

    def _compute(b):
        rows = rbuf[b]

        def _accum(i, _):
            for u in range(4):
                dsl = pl.ds((i * 4 + u) * _LANES, _LANES)
                a = acc[0, dsl]
                for t in range(_TSUB):
                    a = a + rows[t, dsl] * fv[t, dsl]
                acc[0, dsl] = a
            return 0

        lax.fori_loop(0, _D // (_LANES * 4), _accum, 0)

    _start_f(0)
    _start_g(0, 0)

    def _pair(i, _):
        _start_g(2 * i + 1, 1)
        _wait_f()
        _wait_g(0)
        _compute(0)
        _start_f(2 * i + 1)
        _start_g(2 * i + 2, 0)
        _wait_f()
        _wait_g(1)
        _compute(1)
        _start_f(2 * i + 2)
        return 0

    lax.fori_loop(0, _NSUB // 2, _pair, 0)
    _wait_f()
    _wait_g(0)

    pltpu.sync_copy(acc.at[pl.ds(0, 1)], out_hbm.at[pl.ds(c0 * 4 + tq, 1)])


def _tc_body(in_ref, L_ref, F_ref, out_ref, Lbf_ref, Gm_ref, Fo_ref):
    c = pl.program_id(0)

    @pl.when(c == 0)
    def _():
        Lbf_ref[...] = L_ref[...].astype(jnp.bfloat16)
        fo = F_ref[:, 1, :] * _W
        Fo_ref[...] = fo
        Gm_ref[...] = F_ref[:, 0, :] - fo

    idx_e = _quant(in_ref[0, :, 0:1])
    idx_o = _quant(in_ref[0, :, 1:2])
    lvl = jax.lax.broadcasted_iota(jnp.int32, (_P, _NUM_LEVELS), 1)
    oh = (idx_e == lvl).astype(jnp.bfloat16) + (idx_o == lvl).astype(
        jnp.bfloat16
    ) * jnp.bfloat16(1.0 / _W)
    g = jnp.dot(oh, Lbf_ref[...], preferred_element_type=jnp.float32)
    mask = g > 0
    s = jnp.sum(jnp.where(mask, Gm_ref[...], -Gm_ref[...]) + g * Fo_ref[...],
                axis=0, keepdims=True)
    out_ref[0] = jnp.where(s > 0, 1.0, -1.0)


def _fin_body(p_ref, q_ref, out_ref):
    s = jnp.sum(p_ref[...], axis=1)
    q_sc = jnp.where(s > 0, 1.0, -1.0)
    qa = jnp.concatenate([q_sc, q_ref[...]], axis=0)
    r3 = _roll_lanes(qa, 3)
    r2 = _roll_lanes(qa, 2)
    r1 = _roll_lanes(qa, 1)
    w = (r3[0 : _C - 3] * r2[1 : _C - 2]) * (r1[2 : _C - 1] * qa[3:_C])
    s2 = jnp.sum(w, axis=0, keepdims=True)
    out_ref[...] = jnp.where(s2 > 0, 1.0, -1.0)


@jax.jit
def kernel(input, level_weight, features_weight):
    idx = pl.pallas_call(
        _idx_body,
        out_shape=jax.ShapeDtypeStruct((_C_SC, _T), jnp.int32),
    )(input[:_C_SC])
    idx_flat = jnp.reshape(idx, (_C_SC * _T,))
    partials = _sc_stage(idx_flat, level_weight, features_weight)

    x3 = jnp.reshape(input[_C_SC:], (_C_TC, _P, 2))
    F3 = jnp.reshape(features_weight, (_P, 2, _D))
    q_tc = pl.pallas_call(
        _tc_body,
        grid=(_C_TC,),
        in_specs=[
            pl.BlockSpec((1, _P, 2), lambda c: (c, 0, 0)),
            pl.BlockSpec((_NUM_LEVELS, _D), lambda c: (0, 0)),
            pl.BlockSpec((_P, 2, _D), lambda c: (0, 0, 0)),
        ],
        out_specs=pl.BlockSpec((1, 1, _D), lambda c: (c, 0, 0)),
        out_shape=jax.ShapeDtypeStruct((_C_TC, 1, _D), jnp.float32),
        scratch_shapes=[
            pltpu.VMEM((_NUM_LEVELS, _D), jnp.bfloat16),
            pltpu.VMEM((_P, _D), jnp.float32),
            pltpu.VMEM((_P, _D), jnp.float32),
        ],
    )(x3, level_weight, F3)

    p4 = jnp.reshape(partials, (_C_SC, 4, _D))
    out = pl.pallas_call(
        _fin_body,
        out_shape=jax.ShapeDtypeStruct((1, _D), jnp.float32),
    )(p4, jnp.reshape(q_tc, (_C_TC, _D)))
    return out


# speedup vs baseline: 2.6027x; 1.2752x over previous
import functools

import jax
import jax.numpy as jnp
from jax import lax
from jax.experimental import pallas as pl
from jax.experimental.pallas import tpu as pltpu
from jax.experimental.pallas import tpu_sc as plsc

_MAX_VAL = 52000.0
_MIN_VAL = -53000.0
_NUM_LEVELS = 1000
_C = 24
_T = 256
_C_SC = 4
_NSPLIT = 32 // _C_SC
_TTASK = _T // _NSPLIT
_C_TC = _C - _C_SC
_P = _T // 2
_D = 4096
_W = 128.0
_LANES = 16
_TSUB = 8
_NSUB = 4


def _quant(x):
    y = (x - _MIN_VAL) / (_MAX_VAL - _MIN_VAL) * (_NUM_LEVELS - 1)
    return jnp.clip(jnp.round(y), 0, _NUM_LEVELS - 1).astype(jnp.int32)


def _roll_lanes(x, shift):
    return jnp.concatenate([x[:, -shift:], x[:, :-shift]], axis=1)


def _idx_body(in_ref, out_ref):
    out_ref[...] = _quant(in_ref[...])


@functools.partial(
    pl.kernel,
    mesh=plsc.VectorSubcoreMesh(core_axis_name="c", subcore_axis_name="s"),
    out_type=jax.ShapeDtypeStruct((_C_SC * _NSPLIT, _D), jnp.float32),
    scratch_types=[
        pltpu.VMEM((_TTASK,), jnp.int32),
        pltpu.VMEM((_TSUB, _D), jnp.float32),
        pltpu.VMEM((_TSUB, _D), jnp.float32),
        pltpu.VMEM((_TSUB, _D), jnp.float32),
        pltpu.VMEM((1, _D), jnp.float32),
        pltpu.SemaphoreType.DMA,
        pltpu.SemaphoreType.DMA,
        pltpu.SemaphoreType.DMA,
    ],
)
def _sc_stage(idx_hbm, L_hbm, F_hbm, out_hbm, idxv, fv, r0, r1, acc,
              sf, sg0, sg1):
    wid = lax.axis_index("s") * 2 + lax.axis_index("c")
    c0 = lax.rem(wid, _C_SC)
    tq = wid // _C_SC
    t0 = tq * _TTASK

    pltpu.sync_copy(idx_hbm.at[pl.ds(c0 * _T + t0, _TTASK)], idxv)

    zero = jnp.zeros((_LANES,), jnp.float32)

    def _zero_body(i, _):
        acc[0, pl.ds(i * _LANES, _LANES)] = zero
        return 0

    lax.fori_loop(0, _D // _LANES, _zero_body, 0)

    rbuf = (r0, r1)
    gsem = (sg0, sg1)

    def _start_g(k, b):
        ks = jnp.minimum(k, _NSUB - 1)
        pltpu.async_copy(
            L_hbm.at[idxv.at[pl.ds(ks * _TSUB, _TSUB)]], rbuf[b], gsem[b]
        )

    def _start_f(k):
        ks = jnp.minimum(k, _NSUB - 1)
        pltpu.async_copy(F_hbm.at[pl.ds(t0 + ks * _TSUB, _TSUB)], fv, sf)

    def _wait_g(b):
        pltpu.make_async_copy(
            L_hbm.at[idxv.at[pl.ds(0, _TSUB)]], rbuf[b], gsem[b]
        ).wait()

    def _wait_f():
        pltpu.make_async_copy(F_hbm.at[pl.ds(t0, _TSUB)], fv, sf).wait()

    def _compute(b):
        rows = rbuf[b]

        def _accum(i, _):
            for u in range(4):
                dsl = pl.ds((i * 4 + u) * _LANES, _LANES)
                a = acc[0, dsl]
                for t in range(_TSUB):
                    a = a + rows[t, dsl] * fv[t, dsl]
                acc[0, dsl] = a
            return 0

        lax.fori_loop(0, _D // (_LANES * 4), _accum, 0)

    _start_f(0)
    _start_g(0, 0)

    def _pair(i, _):
        _start_g(2 * i + 1, 1)
        _wait_f()
        _wait_g(0)
        _compute(0)
        _start_f(2 * i + 1)
        _start_g(2 * i + 2, 0)
        _wait_f()
        _wait_g(1)
        _compute(1)
        _start_f(2 * i + 2)
        return 0

    lax.fori_loop(0, _NSUB // 2, _pair, 0)
    _wait_f()
    _wait_g(0)

    pltpu.sync_copy(acc.at[pl.ds(0, 1)], out_hbm.at[pl.ds(c0 * _NSPLIT + tq, 1)])


def _tc_body(in_ref, L_ref, F_ref, out_ref, Lbf_ref, Gm_ref, Fo_ref):
    c = pl.program_id(0)

    @pl.when(c == 0)
    def _():
        Lbf_ref[...] = L_ref[...].astype(jnp.bfloat16)
        fo = F_ref[:, 1, :] * _W
        Fo_ref[...] = fo
        Gm_ref[...] = F_ref[:, 0, :] - fo

    idx_e = _quant(in_ref[0, :, 0:1])
    idx_o = _quant(in_ref[0, :, 1:2])
    lvl = jax.lax.broadcasted_iota(jnp.int32, (_P, _NUM_LEVELS), 1)
    oh = (idx_e == lvl).astype(jnp.bfloat16) + (idx_o == lvl).astype(
        jnp.bfloat16
    ) * jnp.bfloat16(1.0 / _W)
    g = jnp.dot(oh, Lbf_ref[...], preferred_element_type=jnp.float32)
    mask = g > 0
    s = jnp.sum(jnp.where(mask, Gm_ref[...], -Gm_ref[...]) + g * Fo_ref[...],
                axis=0, keepdims=True)
    out_ref[0] = jnp.where(s > 0, 1.0, -1.0)


def _fin_body(p_ref, q_ref, out_ref):
    s = jnp.sum(p_ref[...], axis=1)
    q_sc = jnp.where(s > 0, 1.0, -1.0)
    qa = jnp.concatenate([q_sc, q_ref[...]], axis=0)
    r3 = _roll_lanes(qa, 3)
    r2 = _roll_lanes(qa, 2)
    r1 = _roll_lanes(qa, 1)
    w = (r3[0 : _C - 3] * r2[1 : _C - 2]) * (r1[2 : _C - 1] * qa[3:_C])
    s2 = jnp.sum(w, axis=0, keepdims=True)
    out_ref[...] = jnp.where(s2 > 0, 1.0, -1.0)


@jax.jit
def kernel(input, level_weight, features_weight):
    idx = pl.pallas_call(
        _idx_body,
        out_shape=jax.ShapeDtypeStruct((_C_SC, _T), jnp.int32),
    )(input[:_C_SC])
    idx_flat = jnp.reshape(idx, (_C_SC * _T,))
    partials = _sc_stage(idx_flat, level_weight, features_weight)

    x3 = jnp.reshape(input[_C_SC:], (_C_TC, _P, 2))
    F3 = jnp.reshape(features_weight, (_P, 2, _D))
    q_tc = pl.pallas_call(
        _tc_body,
        grid=(_C_TC,),
        in_specs=[
            pl.BlockSpec((1, _P, 2), lambda c: (c, 0, 0)),
            pl.BlockSpec((_NUM_LEVELS, _D), lambda c: (0, 0)),
            pl.BlockSpec((_P, 2, _D), lambda c: (0, 0, 0)),
        ],
        out_specs=pl.BlockSpec((1, 1, _D), lambda c: (c, 0, 0)),
        out_shape=jax.ShapeDtypeStruct((_C_TC, 1, _D), jnp.float32),
        scratch_shapes=[
            pltpu.VMEM((_NUM_LEVELS, _D), jnp.bfloat16),
            pltpu.VMEM((_P, _D), jnp.float32),
            pltpu.VMEM((_P, _D), jnp.float32),
        ],
    )(x3, level_weight, F3)

    p4 = jnp.reshape(partials, (_C_SC, _NSPLIT, _D))
    out = pl.pallas_call(
        _fin_body,
        out_shape=jax.ShapeDtypeStruct((1, _D), jnp.float32),
    )(p4, jnp.reshape(q_tc, (_C_TC, _D)))
    return out


# speedup vs baseline: 3.6296x; 1.3945x over previous
import jax
import jax.numpy as jnp
from jax.experimental import pallas as pl
from jax.experimental.pallas import tpu as pltpu

_MAX_VAL = 52000.0
_MIN_VAL = -53000.0
_NUM_LEVELS = 1000
_LEVELS_PAD = 1024
_C = 24
_T = 256
_P = _T // 2
_D = 4096
_W = 128.0


def _roll_lanes(x, shift):
    return jnp.concatenate([x[:, -shift:], x[:, :-shift]], axis=1)


def _quant(x):
    y = (x - _MIN_VAL) / (_MAX_VAL - _MIN_VAL) * (_NUM_LEVELS - 1)
    return jnp.clip(jnp.round(y), 0, _NUM_LEVELS - 1).astype(jnp.int32)


def _body(in_ref, L_ref, F_ref, out_ref, Lbf_ref, Gm_ref, Fo_ref, smp_ref):
    c = pl.program_id(0)

    @pl.when(c == 0)
    def _():
        Lbf_ref[...] = L_ref[...].astype(jnp.bfloat16)
        fo = F_ref[:, 1, :] * _W
        Fo_ref[...] = fo
        Gm_ref[...] = F_ref[:, 0, :] - fo

    lvl = jax.lax.broadcasted_iota(jnp.int32, (_P, _NUM_LEVELS), 1)

    def _packed_onehot(ch):
        idx_e = _quant(in_ref[ch, :, 0:1])
        idx_o = _quant(in_ref[ch, :, 1:2])
        return (idx_e == lvl).astype(jnp.bfloat16) + (idx_o == lvl).astype(
            jnp.bfloat16
        ) * jnp.bfloat16(1.0 / _W)

    oh = jnp.concatenate([_packed_onehot(0), _packed_onehot(1)], axis=0)
    g = jnp.dot(oh, Lbf_ref[...], preferred_element_type=jnp.float32)
    mask = g > 0
    gm = Gm_ref[...]
    fo = Fo_ref[...]
    t0 = jnp.where(mask[:_P], gm, -gm) + g[:_P] * fo
    t1 = jnp.where(mask[_P:], gm, -gm) + g[_P:] * fo
    s0 = jnp.sum(t0, axis=0, keepdims=True)
    s1 = jnp.sum(t1, axis=0, keepdims=True)
    smp_ref[pl.ds(2 * c, 1), :] = jnp.where(s0 > 0, 1.0, -1.0)
    smp_ref[pl.ds(2 * c + 1, 1), :] = jnp.where(s1 > 0, 1.0, -1.0)

    @pl.when(c == _C // 2 - 1)
    def _():
        qa = smp_ref[...]
        r3 = _roll_lanes(qa, 3)
        r2 = _roll_lanes(qa, 2)
        r1 = _roll_lanes(qa, 1)
        w = (r3[0 : _C - 3] * r2[1 : _C - 2]) * (r1[2 : _C - 1] * qa[3:_C])
        s2 = jnp.sum(w, axis=0, keepdims=True)
        out_ref[...] = jnp.where(s2 > 0, 1.0, -1.0)


@jax.jit
def kernel(input, level_weight, features_weight):
    x3 = jnp.reshape(input, (_C, _P, 2))
    F3 = jnp.reshape(features_weight, (_P, 2, _D))
    out = pl.pallas_call(
        _body,
        grid=(_C // 2,),
        in_specs=[
            pl.BlockSpec((2, _P, 2), lambda c: (c, 0, 0)),
            pl.BlockSpec((_NUM_LEVELS, _D), lambda c: (0, 0)),
            pl.BlockSpec((_P, 2, _D), lambda c: (0, 0, 0)),
        ],
        out_specs=pl.BlockSpec((1, _D), lambda c: (0, 0)),
        out_shape=jax.ShapeDtypeStruct((1, _D), jnp.float32),
        scratch_shapes=[
            pltpu.VMEM((_NUM_LEVELS, _D), jnp.bfloat16),
            pltpu.VMEM((_P, _D), jnp.float32),
            pltpu.VMEM((_P, _D), jnp.float32),
            pltpu.VMEM((_C, _D), jnp.float32),
        ],
    )(x3, level_weight, F3)
    return out


# speedup vs baseline: 3.8079x; 1.0491x over previous
import jax
import jax.numpy as jnp
from jax.experimental import pallas as pl
from jax.experimental.pallas import tpu as pltpu

_MAX_VAL = 52000.0
_MIN_VAL = -53000.0
_NUM_LEVELS = 1000
_LEVELS_PAD = 1024
_C = 24
_T = 256
_P = _T // 2
_D = 4096
_W = 128.0
_CB = 4


def _roll_lanes(x, shift):
    return jnp.concatenate([x[:, -shift:], x[:, :-shift]], axis=1)


def _quant(x):
    y = (x - _MIN_VAL) / (_MAX_VAL - _MIN_VAL) * (_NUM_LEVELS - 1)
    return jnp.clip(jnp.round(y), 0, _NUM_LEVELS - 1).astype(jnp.int32)


def _body(in_ref, L_ref, F_ref, out_ref, Lbf_ref, Gm_ref, Fo_ref, smp_ref):
    c = pl.program_id(0)

    @pl.when(c == 0)
    def _():
        Lbf_ref[...] = L_ref[...].astype(jnp.bfloat16)
        fo = F_ref[:, 1, :] * _W
        Fo_ref[...] = fo
        Gm_ref[...] = F_ref[:, 0, :] - fo

    lvl = jax.lax.broadcasted_iota(jnp.int32, (_P, _NUM_LEVELS), 1)

    def _packed_onehot(ch):
        idx_e = _quant(in_ref[ch, :, 0:1])
        idx_o = _quant(in_ref[ch, :, 1:2])
        return (idx_e == lvl).astype(jnp.bfloat16) + (idx_o == lvl).astype(
            jnp.bfloat16
        ) * jnp.bfloat16(1.0 / _W)

    oh = jnp.concatenate([_packed_onehot(i) for i in range(_CB)], axis=0)
    g = jnp.dot(oh, Lbf_ref[...], preferred_element_type=jnp.float32)
    mask = g > 0
    gm = Gm_ref[...]
    fo = Fo_ref[...]
    for i in range(_CB):
        sl = slice(i * _P, (i + 1) * _P)
        ti = jnp.where(mask[sl], gm, -gm) + g[sl] * fo
        si = jnp.sum(ti, axis=0, keepdims=True)
        smp_ref[pl.ds(_CB * c + i, 1), :] = jnp.where(si > 0, 1.0, -1.0)

    @pl.when(c == _C // _CB - 1)
    def _():
        qa = smp_ref[...]
        r3 = _roll_lanes(qa, 3)
        r2 = _roll_lanes(qa, 2)
        r1 = _roll_lanes(qa, 1)
        w = (r3[0 : _C - 3] * r2[1 : _C - 2]) * (r1[2 : _C - 1] * qa[3:_C])
        s2 = jnp.sum(w, axis=0, keepdims=True)
        out_ref[...] = jnp.where(s2 > 0, 1.0, -1.0)


@jax.jit
def kernel(input, level_weight, features_weight):
    x3 = jnp.reshape(input, (_C, _P, 2))
    F3 = jnp.reshape(features_weight, (_P, 2, _D))
    out = pl.pallas_call(
        _body,
        grid=(_C // _CB,),
        in_specs=[
            pl.BlockSpec((_CB, _P, 2), lambda c: (c, 0, 0)),
            pl.BlockSpec((_NUM_LEVELS, _D), lambda c: (0, 0)),
            pl.BlockSpec((_P, 2, _D), lambda c: (0, 0, 0)),
        ],
        out_specs=pl.BlockSpec((1, _D), lambda c: (0, 0)),
        out_shape=jax.ShapeDtypeStruct((1, _D), jnp.float32),
        scratch_shapes=[
            pltpu.VMEM((_NUM_LEVELS, _D), jnp.bfloat16),
            pltpu.VMEM((_P, _D), jnp.float32),
            pltpu.VMEM((_P, _D), jnp.float32),
            pltpu.VMEM((_C, _D), jnp.float32),
        ],
    )(x3, level_weight, F3)
    return out


# speedup vs baseline: 3.8899x; 1.0215x over previous
import jax
import jax.numpy as jnp
from jax.experimental import pallas as pl
from jax.experimental.pallas import tpu as pltpu

_MAX_VAL = 52000.0
_MIN_VAL = -53000.0
_NUM_LEVELS = 1000
_LEVELS_PAD = 1024
_C = 24
_T = 256
_P = _T // 2
_D = 4096
_W = 128.0
_CB = 8


def _roll_lanes(x, shift):
    return jnp.concatenate([x[:, -shift:], x[:, :-shift]], axis=1)


def _quant(x):
    y = (x - _MIN_VAL) / (_MAX_VAL - _MIN_VAL) * (_NUM_LEVELS - 1)
    return jnp.clip(jnp.round(y), 0, _NUM_LEVELS - 1).astype(jnp.int32)


def _body(in_ref, L_ref, F_ref, out_ref, Lbf_ref, Gm_ref, Fo_ref, smp_ref):
    c = pl.program_id(0)

    @pl.when(c == 0)
    def _():
        Lbf_ref[...] = L_ref[...].astype(jnp.bfloat16)
        fo = F_ref[:, 1, :] * _W
        Fo_ref[...] = fo
        Gm_ref[...] = F_ref[:, 0, :] - fo

    lvl = jax.lax.broadcasted_iota(jnp.int32, (_P, _NUM_LEVELS), 1)

    def _packed_onehot(ch):
        idx_e = _quant(in_ref[ch, :, 0:1])
        idx_o = _quant(in_ref[ch, :, 1:2])
        return (idx_e == lvl).astype(jnp.bfloat16) + (idx_o == lvl).astype(
            jnp.bfloat16
        ) * jnp.bfloat16(1.0 / _W)

    oh = jnp.concatenate([_packed_onehot(i) for i in range(_CB)], axis=0)
    g = jnp.dot(oh, Lbf_ref[...], preferred_element_type=jnp.float32)
    mask = g > 0
    gm = Gm_ref[...]
    fo = Fo_ref[...]
    for i in range(_CB):
        sl = slice(i * _P, (i + 1) * _P)
        ti = jnp.where(mask[sl], gm, -gm) + g[sl] * fo
        si = jnp.sum(ti, axis=0, keepdims=True)
        smp_ref[pl.ds(_CB * c + i, 1), :] = jnp.where(si > 0, 1.0, -1.0)

    @pl.when(c == _C // _CB - 1)
    def _():
        qa = smp_ref[...]
        r3 = _roll_lanes(qa, 3)
        r2 = _roll_lanes(qa, 2)
        r1 = _roll_lanes(qa, 1)
        w = (r3[0 : _C - 3] * r2[1 : _C - 2]) * (r1[2 : _C - 1] * qa[3:_C])
        s2 = jnp.sum(w, axis=0, keepdims=True)
        out_ref[...] = jnp.where(s2 > 0, 1.0, -1.0)


@jax.jit
def kernel(input, level_weight, features_weight):
    x3 = jnp.reshape(input, (_C, _P, 2))
    F3 = jnp.reshape(features_weight, (_P, 2, _D))
    out = pl.pallas_call(
        _body,
        grid=(_C // _CB,),
        in_specs=[
            pl.BlockSpec((_CB, _P, 2), lambda c: (c, 0, 0)),
            pl.BlockSpec((_NUM_LEVELS, _D), lambda c: (0, 0)),
            pl.BlockSpec((_P, 2, _D), lambda c: (0, 0, 0)),
        ],
        out_specs=pl.BlockSpec((1, _D), lambda c: (0, 0)),
        out_shape=jax.ShapeDtypeStruct((1, _D), jnp.float32),
        scratch_shapes=[
            pltpu.VMEM((_NUM_LEVELS, _D), jnp.bfloat16),
            pltpu.VMEM((_P, _D), jnp.float32),
            pltpu.VMEM((_P, _D), jnp.float32),
            pltpu.VMEM((_C, _D), jnp.float32),
        ],
    )(x3, level_weight, F3)
    return out
